# baseline (speedup 1.0000x reference)
import jax
import jax.numpy as jnp
from jax.experimental import pallas as pl
from jax.experimental.pallas import tpu as pltpu

_B, _S, _D, _H, _E, _K = 32, 4, 768, 256, 8, 2
_EPS_NORM = 1e-6


def _moe_kernel(x_ref, nw_ref, gwt_ref, w1_ref, w2_ref, w3_ref, out_ref):
    x = x_ref[...]
    nw = nw_ref[...]
    xn = x * jax.lax.rsqrt(jnp.mean(x * x, axis=-1, keepdims=True) + _EPS_NORM)
    xn = xn * nw

    logits = jnp.dot(xn, gwt_ref[...], preferred_element_type=jnp.float32)
    p = jax.nn.softmax(logits, axis=-1)
    iota = jax.lax.broadcasted_iota(jnp.int32, p.shape, 1)
    m1 = jnp.max(p, axis=-1, keepdims=True)
    i1 = jnp.min(jnp.where(p >= m1, iota, _E), axis=-1, keepdims=True)
    one1 = iota == i1
    p2 = jnp.where(one1, -1.0, p)
    m2 = jnp.max(p2, axis=-1, keepdims=True)
    i2 = jnp.min(jnp.where(p2 >= m2, iota, _E), axis=-1, keepdims=True)
    one2 = iota == i2
    c = jnp.where(one1 | one2, p, 0.0) / (m1 + m2 + 1e-10)

    xnb = xn.astype(jnp.bfloat16)
    acc = jnp.zeros(out_ref.shape, jnp.float32)
    for e in range(_E):
        h1 = jnp.dot(xnb, w1_ref[e], preferred_element_type=jnp.float32)
        h2 = jnp.dot(xnb, w2_ref[e], preferred_element_type=jnp.float32)
        hid = ((h1 * jax.lax.logistic(h1)) * h2).astype(jnp.bfloat16)
        oe = jnp.dot(hid, w3_ref[e], preferred_element_type=jnp.float32)
        acc = acc + c[:, e:e + 1] * oe
    out_ref[...] = acc


def kernel(x, norm_weight, gate_w, w1, w2, w3):
    b, s, d = x.shape
    t = b * s
    x_flat = x.reshape(t, d)
    nw = norm_weight.reshape(1, d)
    gwt = gate_w.T
    out = pl.pallas_call(
        _moe_kernel,
        out_shape=jax.ShapeDtypeStruct((t, d), jnp.float32),
    )(x_flat, nw, gwt,
      w1.astype(jnp.bfloat16), w2.astype(jnp.bfloat16),
      w3.astype(jnp.bfloat16))
    return out.reshape(b, s, d)


# speedup vs baseline: 1.4001x; 1.4001x over previous
import jax
import jax.numpy as jnp
from jax.experimental import pallas as pl
from jax.experimental.pallas import tpu as pltpu

_B, _S, _D, _H, _E, _K = 32, 4, 768, 256, 8, 2
_EPS_NORM = 1e-6


def _moe_kernel(x_ref, nw_ref, gwt_ref, w1_ref, w2_ref, w3_ref, out_ref):
    x = x_ref[...]
    nw = nw_ref[...]
    xn = x * jax.lax.rsqrt(jnp.mean(x * x, axis=-1, keepdims=True) + _EPS_NORM)
    xn = xn * nw

    logits = jnp.dot(xn, gwt_ref[...], preferred_element_type=jnp.float32)
    p = jax.nn.softmax(logits, axis=-1)
    iota = jax.lax.broadcasted_iota(jnp.int32, p.shape, 1)
    m1 = jnp.max(p, axis=-1, keepdims=True)
    i1 = jnp.min(jnp.where(p >= m1, iota, _E), axis=-1, keepdims=True)
    one1 = iota == i1
    p2 = jnp.where(one1, -1.0, p)
    m2 = jnp.max(p2, axis=-1, keepdims=True)
    i2 = jnp.min(jnp.where(p2 >= m2, iota, _E), axis=-1, keepdims=True)
    one2 = iota == i2
    c = jnp.where(one1 | one2, p, 0.0) / (m1 + m2 + 1e-10)

    acc = jnp.zeros(out_ref.shape, jnp.float32)
    for e in range(_E):
        h1 = jnp.dot(xn, w1_ref[e], preferred_element_type=jnp.float32)
        h2 = jnp.dot(xn, w2_ref[e], preferred_element_type=jnp.float32)
        hid = (h1 * jax.lax.logistic(h1)) * h2
        oe = jnp.dot(hid, w3_ref[e], preferred_element_type=jnp.float32)
        acc = acc + c[:, e:e + 1] * oe
    out_ref[...] = acc


def kernel(x, norm_weight, gate_w, w1, w2, w3):
    b, s, d = x.shape
    t = b * s
    x_flat = x.reshape(t, d)
    nw = norm_weight.reshape(1, d)
    gwt = gate_w.T
    out = pl.pallas_call(
        _moe_kernel,
        out_shape=jax.ShapeDtypeStruct((t, d), jnp.float32),
    )(x_flat, nw, gwt, w1, w2, w3)
    return out.reshape(b, s, d)


# speedup vs baseline: 1.4324x; 1.0231x over previous
import jax
import jax.numpy as jnp
from jax.experimental import pallas as pl
from jax.experimental.pallas import tpu as pltpu

_B, _S, _D, _H, _E, _K = 32, 4, 768, 256, 8, 2
_EPS_NORM = 1e-6


def _moe_kernel(x_ref, nw_ref, gwt_ref, w1_ref, w2_ref, w3_ref, out_ref):
    x = x_ref[...]
    nw = nw_ref[...]
    xn = x * jax.lax.rsqrt(jnp.mean(x * x, axis=-1, keepdims=True) + _EPS_NORM)
    xn = xn * nw

    logits = jnp.dot(xn, gwt_ref[...], preferred_element_type=jnp.float32)
    p = jax.nn.softmax(logits, axis=-1)
    iota = jax.lax.broadcasted_iota(jnp.int32, p.shape, 1)
    m1 = jnp.max(p, axis=-1, keepdims=True)
    i1 = jnp.min(jnp.where(p >= m1, iota, _E), axis=-1, keepdims=True)
    one1 = iota == i1
    p2 = jnp.where(one1, -1.0, p)
    m2 = jnp.max(p2, axis=-1, keepdims=True)
    i2 = jnp.min(jnp.where(p2 >= m2, iota, _E), axis=-1, keepdims=True)
    one2 = iota == i2
    c = jnp.where(one1 | one2, p, 0.0) / (m1 + m2 + 1e-10)

    xnb = xn.astype(jnp.bfloat16)
    acc = jnp.zeros(out_ref.shape, jnp.float32)
    for e in range(_E):
        w1e = w1_ref[e].astype(jnp.bfloat16)
        w2e = w2_ref[e].astype(jnp.bfloat16)
        w3e = w3_ref[e].astype(jnp.bfloat16)
        h1 = jnp.dot(xnb, w1e, preferred_element_type=jnp.float32)
        h2 = jnp.dot(xnb, w2e, preferred_element_type=jnp.float32)
        hid = ((h1 * jax.lax.logistic(h1)) * h2).astype(jnp.bfloat16)
        oe = jnp.dot(hid, w3e, preferred_element_type=jnp.float32)
        acc = acc + c[:, e:e + 1] * oe
    out_ref[...] = acc


def kernel(x, norm_weight, gate_w, w1, w2, w3):
    b, s, d = x.shape
    t = b * s
    x_flat = x.reshape(t, d)
    nw = norm_weight.reshape(1, d)
    gwt = gate_w.T
    out = pl.pallas_call(
        _moe_kernel,
        out_shape=jax.ShapeDtypeStruct((t, d), jnp.float32),
    )(x_flat, nw, gwt, w1, w2, w3)
    return out.reshape(b, s, d)
